# Initial kernel scaffold; baseline (speedup 1.0000x reference)
import jax
import jax.numpy as jnp
from jax.experimental import pallas as pl


def kernel(inputs, channels):
    raise NotImplementedError("write your pallas kernel here")


# speedup vs baseline: 1.0459x; 1.0459x over previous
import functools

import jax
import jax.numpy as jnp
from jax import lax
from jax.experimental import pallas as pl
from jax.experimental.pallas import tpu as pltpu
from jax.experimental.pallas import tpu_sc as plsc

_B = 32
_CIN = 768
_COUT = 384
_D = 1024
_NC = 2
_NS = 16
_NW = _NC * _NS
_CH = 48
_NCHUNK = _COUT // _CH

_mesh = plsc.VectorSubcoreMesh(core_axis_name="c", subcore_axis_name="s")


@functools.partial(
    pl.kernel,
    mesh=_mesh,
    out_type=jax.ShapeDtypeStruct((_B * _COUT, _D), jnp.float32),
    scratch_types=[
        pltpu.VMEM((_NCHUNK, _CH), jnp.int32),
        pltpu.VMEM((_CH, _D), jnp.float32),
        pltpu.VMEM((_CH, _D), jnp.float32),
        pltpu.SemaphoreType.DMA,
        pltpu.SemaphoreType.DMA,
        pltpu.SemaphoreType.DMA,
        pltpu.SemaphoreType.DMA,
    ],
)
def _gather_rows(tbl, idx_hbm, out, idx_v, buf0, buf1, g0, g1, s0, s1):
    wid = lax.axis_index("s") * _NC + lax.axis_index("c")
    base = wid * _COUT
    pltpu.sync_copy(idx_hbm.at[wid], idx_v)
    bufs = (buf0, buf1)
    gsems = (g0, g1)
    ssems = (s0, s1)
    gathers = [None] * _NCHUNK
    scatters = [None] * _NCHUNK
    gathers[0] = pltpu.async_copy(tbl.at[idx_v.at[0]], bufs[0], gsems[0])
    gathers[1] = pltpu.async_copy(tbl.at[idx_v.at[1]], bufs[1], gsems[1])
    for c in range(_NCHUNK):
        p = c % 2
        gathers[c].wait()
        scatters[c] = pltpu.async_copy(
            bufs[p], out.at[pl.ds(base + c * _CH, _CH)], ssems[p])
        if c + 2 < _NCHUNK:
            scatters[c].wait()
            gathers[c + 2] = pltpu.async_copy(
                tbl.at[idx_v.at[c + 2]], bufs[p], gsems[p])
    scatters[_NCHUNK - 2].wait()
    scatters[_NCHUNK - 1].wait()


def kernel(inputs, channels):
    tbl = inputs.reshape(_B * _CIN, _D)
    row_idx = (
        jnp.arange(_B, dtype=jnp.int32)[:, None] * _CIN
        + channels.astype(jnp.int32)[None, :]
    ).reshape(_B, _NCHUNK, _CH)
    out = _gather_rows(tbl, row_idx)
    return out.reshape(_B, _COUT, 32, 32)
